# baseline (speedup 1.0000x reference)
import jax
import jax.numpy as jnp
from jax import lax
from jax.experimental import pallas as pl
from jax.experimental.pallas import tpu as pltpu
from jax.experimental.pallas import tpu_sc as plsc

B = 4096
T = 200
D = 64
VOCAB = 1000000

NC = 2
NS = 16
TR = T // 8
CB = B // 128


def _sc_body(tok_hbm, table_hbm, pos_hbm, out_hbm,
             pos_v, idx2, grow2, sbuf2, gsem0, gsem1, osem0, osem1):
    wid = lax.axis_index("s") * NC + lax.axis_index("c")
    gsems = (gsem0, gsem1)
    osems = (osem0, osem1)

    pltpu.sync_copy(pos_hbm, pos_v)

    iotas = [lax.iota(jnp.int32, 16) + 16 * q for q in range(4)]

    def start_gather(t, slot):
        tr = lax.shift_right_logical(t, 3)
        s = lax.bitwise_and(t, 7)
        pltpu.sync_copy(tok_hbm.at[tr, wid, s], idx2.at[slot])
        pltpu.make_async_copy(
            table_hbm.at[idx2.at[slot]], grow2.at[slot], gsems[slot]
        ).start()

    def wait_gather(slot):
        pltpu.make_async_copy(
            table_hbm.at[idx2.at[slot]], grow2.at[slot], gsems[slot]
        ).wait()

    def out_copy(t, r, slot):
        return pltpu.make_async_copy(
            sbuf2.at[slot, pl.ds(8 * r, 8)], out_hbm.at[t, r, wid],
            osems[slot],
        )

    start_gather(0, 0)

    def gbody(g, carry):
        for b in range(2):
            t = 2 * g + b
            nt = t + 1

            @pl.when(nt < T)
            def _():
                start_gather(nt, 1 - b)

            wait_gather(b)

            @pl.when(t >= 2)
            def _():
                for r in range(8):
                    out_copy(t - 2, r, b).wait()

            pvec = [pos_v[t, pl.ds(16 * q, 16)] for q in range(4)]
            sb = sbuf2.at[b]
            for j in range(128):
                jf = jnp.full((16,), j, jnp.int32)
                for q in range(4):
                    val = grow2[b, j, pl.ds(16 * q, 16)] + pvec[q]
                    plsc.store_scatter(sb, [iotas[q], jf], val)

            for r in range(8):
                out_copy(t, r, b).start()
        return carry

    lax.fori_loop(0, T // 2, gbody, 0)
    for b, t in ((0, T - 2), (1, T - 1)):
        for r in range(8):
            out_copy(t, r, b).wait()


@jax.jit
def _sc_lookup(tok5, emb_weight, pos):
    mesh = plsc.VectorSubcoreMesh(core_axis_name="c", subcore_axis_name="s")
    fn = pl.kernel(
        _sc_body,
        out_type=jax.ShapeDtypeStruct((T, 8, CB, 8, 128), jnp.float32),
        mesh=mesh,
        scratch_types=[
            pltpu.VMEM((T, D), jnp.float32),
            pltpu.VMEM((2, 128), jnp.int32),
            pltpu.VMEM((2, 128, D), jnp.float32),
            pltpu.VMEM((2, D, 128), jnp.float32),
            pltpu.SemaphoreType.DMA,
            pltpu.SemaphoreType.DMA,
            pltpu.SemaphoreType.DMA,
            pltpu.SemaphoreType.DMA,
        ],
        compiler_params=pltpu.CompilerParams(
            use_tc_tiling_on_sc=False, needs_layout_passes=False
        ),
    )
    return fn(tok5, emb_weight, pos)


def kernel(tokens, emb_weight, pos):
    tok5 = (tokens.astype(jnp.int32)
            .reshape(CB, 128, TR, 8).transpose(2, 0, 3, 1))
    out5 = _sc_lookup(tok5, emb_weight, pos)
    return out5.transpose(2, 4, 0, 1, 3).reshape(B, T, D)


# speedup vs baseline: 2.0784x; 2.0784x over previous
import jax
import jax.numpy as jnp
from jax import lax
from jax.experimental import pallas as pl
from jax.experimental.pallas import tpu as pltpu
from jax.experimental.pallas import tpu_sc as plsc

B = 4096
T = 200
D = 64
VOCAB = 1000000

NC = 2
NS = 16
TR = T // 8
CB = B // 128


def _sc_body(tok_hbm, table_hbm, pos_hbm, out_hbm,
             pos_v, idx2, grow2, sbuf2, gsem0, gsem1, osem0, osem1):
    wid = lax.axis_index("s") * NC + lax.axis_index("c")
    gsems = (gsem0, gsem1)
    osems = (osem0, osem1)

    pltpu.sync_copy(pos_hbm, pos_v)

    iotas = [lax.iota(jnp.int32, 16) + 16 * q for q in range(4)]

    def start_gather(t, slot):
        tr = lax.shift_right_logical(t, 3)
        s = lax.bitwise_and(t, 7)
        pltpu.sync_copy(tok_hbm.at[tr, wid, s], idx2.at[slot])
        pltpu.make_async_copy(
            table_hbm.at[idx2.at[slot]], grow2.at[slot], gsems[slot]
        ).start()

    def wait_gather(slot):
        pltpu.make_async_copy(
            table_hbm.at[idx2.at[slot]], grow2.at[slot], gsems[slot]
        ).wait()

    def out_copy(t, r, slot):
        return pltpu.make_async_copy(
            sbuf2.at[slot, pl.ds(8 * r, 8), pl.ds(0, 128)],
            out_hbm.at[t, r, wid],
            osems[slot],
        )

    start_gather(0, 0)

    def gbody(g, carry):
        for b in range(2):
            t = 2 * g + b
            nt = t + 1

            @pl.when(nt < T)
            def _():
                start_gather(nt, 1 - b)

            wait_gather(b)

            @pl.when(t >= 2)
            def _():
                for r in range(8):
                    out_copy(t - 2, r, b).wait()

            pvec = [pos_v[t, pl.ds(16 * q, 16)] for q in range(4)]
            sb = sbuf2.at[b]

            @plsc.parallel_loop(0, 128, 1, unroll=8)
            def _(j):
                jf = jnp.full((16,), 0, jnp.int32) + j
                for q in range(4):
                    val = grow2[b, j, pl.ds(16 * q, 16)] + pvec[q]
                    plsc.store_scatter(sb, [iotas[q], jf], val)

            for r in range(8):
                out_copy(t, r, b).start()
        return carry

    lax.fori_loop(0, T // 2, gbody, 0)
    for b, t in ((0, T - 2), (1, T - 1)):
        for r in range(8):
            out_copy(t, r, b).wait()


@jax.jit
def _sc_lookup(tok5, emb_weight, pos):
    mesh = plsc.VectorSubcoreMesh(core_axis_name="c", subcore_axis_name="s")
    fn = pl.kernel(
        _sc_body,
        out_type=jax.ShapeDtypeStruct((T, 8, CB, 8, 128), jnp.float32),
        mesh=mesh,
        scratch_types=[
            pltpu.VMEM((T, D), jnp.float32),
            pltpu.VMEM((2, 128), jnp.int32),
            pltpu.VMEM((2, 128, D), jnp.float32),
            pltpu.VMEM((2, D, 133), jnp.float32),
            pltpu.SemaphoreType.DMA,
            pltpu.SemaphoreType.DMA,
            pltpu.SemaphoreType.DMA,
            pltpu.SemaphoreType.DMA,
        ],
        compiler_params=pltpu.CompilerParams(
            use_tc_tiling_on_sc=False, needs_layout_passes=False
        ),
    )
    return fn(tok5, emb_weight, pos)


def kernel(tokens, emb_weight, pos):
    tok5 = (tokens.astype(jnp.int32)
            .reshape(CB, 128, TR, 8).transpose(2, 0, 3, 1))
    out5 = _sc_lookup(tok5, emb_weight, pos)
    return out5.transpose(2, 4, 0, 1, 3).reshape(B, T, D)
